# Initial kernel scaffold; baseline (speedup 1.0000x reference)
import jax
import jax.numpy as jnp
from jax.experimental import pallas as pl


def kernel(dgm):
    raise NotImplementedError("write your pallas kernel here")


# speedup vs baseline: 2.3219x; 2.3219x over previous
import functools

import jax
import jax.numpy as jnp
from jax import lax
from jax.experimental import pallas as pl
from jax.experimental.pallas import tpu as pltpu
from jax.experimental.pallas import tpu_sc as plsc

N = 500000
K = 16
L = 16
NC = 2
NS = 16
NW = NC * NS
PW = 15648
PW_LAST = N - (NW - 1) * PW
ITER_A = PW_LAST // L
ITER_B = (PW - PW_LAST) // L
NEG_INF = float("-inf")

_mesh = plsc.VectorSubcoreMesh(core_axis_name="c", subcore_axis_name="s")
_params = pltpu.CompilerParams(needs_layout_passes=False,
                               use_tc_tiling_on_sc=False)


def _sanitize(lengths):
    lengths = jnp.where(jnp.isinf(lengths), jnp.float32(0.0), lengths)
    return jnp.where(jnp.isnan(lengths), jnp.float32(0.0), lengths)


def _insert_top(tops, v):
    new_tops = []
    for t in tops:
        hi = jnp.maximum(t, v)
        v = jnp.minimum(t, v)
        new_tops.append(hi)
    return new_tops


@functools.partial(
    pl.kernel,
    out_type=(
        jax.ShapeDtypeStruct((NW, K * L), jnp.float32),
        jax.ShapeDtypeStruct((NW, L), jnp.float32),
    ),
    mesh=_mesh,
    compiler_params=_params,
    scratch_types=[
        pltpu.VMEM((2 * PW,), jnp.float32),
        pltpu.VMEM((K * L,), jnp.float32),
        pltpu.VMEM((L,), jnp.float32),
    ],
)
def _scan_kernel(flat_hbm, cand_hbm, psum_hbm, buf, candv, psumv):
    wid = lax.axis_index("s") * NC + lax.axis_index("c")
    base = wid * (2 * PW)

    pltpu.sync_copy(flat_hbm.at[pl.ds(base, 2 * PW_LAST)],
                    buf.at[pl.ds(0, 2 * PW_LAST)])

    @pl.when(wid < NW - 1)
    def _():
        pltpu.sync_copy(flat_hbm.at[pl.ds(base + 2 * PW_LAST,
                                          2 * (PW - PW_LAST))],
                        buf.at[pl.ds(2 * PW_LAST, 2 * (PW - PW_LAST))])

    lane2 = 2 * lax.iota(jnp.int32, L)

    def body(i, carry):
        acc = carry[0]
        tops = list(carry[1:])
        gidx = lane2 + 2 * L * i
        c0 = plsc.load_gather(buf, [gidx])
        c1 = plsc.load_gather(buf, [gidx + 1])
        lengths = _sanitize(c1 - c0)
        acc = acc + lengths * lengths
        tops = _insert_top(tops, lengths)
        return (acc, *tops)

    init = (jnp.zeros((L,), jnp.float32),
            *[jnp.full((L,), NEG_INF, jnp.float32) for _ in range(K)])
    carry = lax.fori_loop(0, ITER_A, body, init)
    carry = lax.cond(wid < NW - 1,
                     lambda c: lax.fori_loop(ITER_A, ITER_A + ITER_B, body, c),
                     lambda c: c,
                     carry)

    psumv[...] = carry[0]
    for k in range(K):
        candv[pl.ds(k * L, L)] = carry[1 + k]
    pltpu.sync_copy(candv, cand_hbm.at[wid])
    pltpu.sync_copy(psumv, psum_hbm.at[wid])


@functools.partial(
    pl.kernel,
    out_type=jax.ShapeDtypeStruct((L,), jnp.float32),
    mesh=_mesh,
    compiler_params=_params,
    scratch_types=[
        pltpu.VMEM((NW * K * L,), jnp.float32),
        pltpu.VMEM((NW * L,), jnp.float32),
        pltpu.VMEM((L,), jnp.float32),
    ],
)
def _merge_kernel(cand_hbm, psum_hbm, out_hbm, cbuf, pbuf, outv):
    wid = lax.axis_index("s") * NC + lax.axis_index("c")

    @pl.when(wid == 0)
    def _():
        pltpu.sync_copy(cand_hbm, cbuf)
        pltpu.sync_copy(psum_hbm, pbuf)

        def sum_body(j, acc):
            return acc + pbuf[pl.ds(j * L, L)]

        totv = lax.fori_loop(0, NW, sum_body,
                             jnp.zeros((L,), jnp.float32))
        total = jnp.sum(totv)

        def top_body(i, carry):
            return tuple(_insert_top(list(carry), cbuf[pl.ds(i * L, L)]))

        init = tuple(jnp.full((L,), NEG_INF, jnp.float32) for _ in range(K))
        tops = lax.fori_loop(0, NW * K, top_body, init)

        top16 = jnp.sort(tops[0])
        for k in range(1, K):
            desc = jnp.flip(jnp.sort(tops[k]))
            top16 = jnp.sort(jnp.maximum(top16, desc))

        result = total - jnp.sum(top16 * top16)
        outv[...] = jnp.full((L,), result, jnp.float32)
        pltpu.sync_copy(outv, out_hbm)


def kernel(dgm):
    flat = jnp.reshape(dgm, (2 * N,))
    cand, psum = _scan_kernel(flat)
    out = _merge_kernel(jnp.reshape(cand, (NW * K * L,)),
                        jnp.reshape(psum, (NW * L,)))
    return out[0]


# speedup vs baseline: 12.3269x; 5.3089x over previous
import functools

import jax
import jax.numpy as jnp
from jax import lax
from jax.experimental import pallas as pl
from jax.experimental.pallas import tpu as pltpu
from jax.experimental.pallas import tpu_sc as plsc

N = 500000
K = 16
L = 16
NC = 2
NS = 16
NW = NC * NS
PW = 15648
PW_LAST = N - (NW - 1) * PW
ITER_A = PW_LAST // L
ITER_B = (PW - PW_LAST) // L
NEG_INF = float("-inf")

_mesh = plsc.VectorSubcoreMesh(core_axis_name="c", subcore_axis_name="s")
_params = pltpu.CompilerParams(needs_layout_passes=False,
                               use_tc_tiling_on_sc=False)


def _sanitize(lengths):
    lengths = jnp.where(jnp.isinf(lengths), jnp.float32(0.0), lengths)
    return jnp.where(jnp.isnan(lengths), jnp.float32(0.0), lengths)


def _insert_top(tops, v):
    new_tops = []
    for t in tops:
        hi = jnp.maximum(t, v)
        v = jnp.minimum(t, v)
        new_tops.append(hi)
    return new_tops


@functools.partial(
    pl.kernel,
    out_type=(
        jax.ShapeDtypeStruct((NW, K * L), jnp.float32),
        jax.ShapeDtypeStruct((NW, L), jnp.float32),
    ),
    mesh=_mesh,
    compiler_params=_params,
    scratch_types=[
        pltpu.VMEM((PW,), jnp.float32),
        pltpu.VMEM((PW,), jnp.float32),
        pltpu.VMEM((K * L,), jnp.float32),
        pltpu.VMEM((L,), jnp.float32),
    ],
)
def _scan_kernel(c0_hbm, c1_hbm, cand_hbm, psum_hbm, buf0, buf1, candv, psumv):
    wid = lax.axis_index("s") * NC + lax.axis_index("c")
    base = wid * PW

    pltpu.sync_copy(c0_hbm.at[pl.ds(base, PW_LAST)], buf0.at[pl.ds(0, PW_LAST)])
    pltpu.sync_copy(c1_hbm.at[pl.ds(base, PW_LAST)], buf1.at[pl.ds(0, PW_LAST)])

    @pl.when(wid < NW - 1)
    def _():
        pltpu.sync_copy(c0_hbm.at[pl.ds(base + PW_LAST, PW - PW_LAST)],
                        buf0.at[pl.ds(PW_LAST, PW - PW_LAST)])
        pltpu.sync_copy(c1_hbm.at[pl.ds(base + PW_LAST, PW - PW_LAST)],
                        buf1.at[pl.ds(PW_LAST, PW - PW_LAST)])

    def body(i, carry):
        acc = carry[0]
        tops = list(carry[1:])
        start = i * L
        lengths = _sanitize(buf1[pl.ds(start, L)] - buf0[pl.ds(start, L)])
        acc = acc + lengths * lengths
        tops = _insert_top(tops, lengths)
        return (acc, *tops)

    init = (jnp.zeros((L,), jnp.float32),
            *[jnp.full((L,), NEG_INF, jnp.float32) for _ in range(K)])
    carry = lax.fori_loop(0, ITER_A, body, init)
    carry = lax.cond(wid < NW - 1,
                     lambda c: lax.fori_loop(ITER_A, ITER_A + ITER_B, body, c),
                     lambda c: c,
                     carry)

    psumv[...] = carry[0]
    for k in range(K):
        candv[pl.ds(k * L, L)] = carry[1 + k]
    pltpu.sync_copy(candv, cand_hbm.at[wid])
    pltpu.sync_copy(psumv, psum_hbm.at[wid])


@functools.partial(
    pl.kernel,
    out_type=jax.ShapeDtypeStruct((L,), jnp.float32),
    mesh=_mesh,
    compiler_params=_params,
    scratch_types=[
        pltpu.VMEM((NW * K * L,), jnp.float32),
        pltpu.VMEM((NW * L,), jnp.float32),
        pltpu.VMEM((L,), jnp.float32),
    ],
)
def _merge_kernel(cand_hbm, psum_hbm, out_hbm, cbuf, pbuf, outv):
    wid = lax.axis_index("s") * NC + lax.axis_index("c")

    @pl.when(wid == 0)
    def _():
        pltpu.sync_copy(cand_hbm, cbuf)
        pltpu.sync_copy(psum_hbm, pbuf)

        def sum_body(j, acc):
            return acc + pbuf[pl.ds(j * L, L)]

        totv = lax.fori_loop(0, NW, sum_body,
                             jnp.zeros((L,), jnp.float32))
        total = jnp.sum(totv)

        def top_body(i, carry):
            return tuple(_insert_top(list(carry), cbuf[pl.ds(i * L, L)]))

        init = tuple(jnp.full((L,), NEG_INF, jnp.float32) for _ in range(K))
        tops = lax.fori_loop(0, NW * K, top_body, init)

        top16 = jnp.sort(tops[0])
        for k in range(1, K):
            desc = jnp.flip(jnp.sort(tops[k]))
            top16 = jnp.sort(jnp.maximum(top16, desc))

        result = total - jnp.sum(top16 * top16)
        outv[...] = jnp.full((L,), result, jnp.float32)
        pltpu.sync_copy(outv, out_hbm)


def kernel(dgm):
    c0 = dgm[:, 0]
    c1 = dgm[:, 1]
    cand, psum = _scan_kernel(c0, c1)
    out = _merge_kernel(jnp.reshape(cand, (NW * K * L,)),
                        jnp.reshape(psum, (NW * L,)))
    return out[0]


# speedup vs baseline: 13.0230x; 1.0565x over previous
import functools

import jax
import jax.numpy as jnp
from jax import lax
from jax.experimental import pallas as pl
from jax.experimental.pallas import tpu as pltpu
from jax.experimental.pallas import tpu_sc as plsc

N = 500000
K = 16
L = 16
NC = 2
NS = 16
NW = NC * NS
PW = 15648
PW_LAST = N - (NW - 1) * PW
ITER_A = PW_LAST // L
ITER_B = (PW - PW_LAST) // L
NEG_INF = float("-inf")

_mesh = plsc.VectorSubcoreMesh(core_axis_name="c", subcore_axis_name="s")
_params = pltpu.CompilerParams(needs_layout_passes=False,
                               use_tc_tiling_on_sc=False)


def _sanitize(lengths):
    lengths = jnp.where(jnp.isinf(lengths), jnp.float32(0.0), lengths)
    return jnp.where(jnp.isnan(lengths), jnp.float32(0.0), lengths)


def _insert_top(tops, v):
    new_tops = []
    for t in tops:
        hi = jnp.maximum(t, v)
        v = jnp.minimum(t, v)
        new_tops.append(hi)
    return new_tops


def _merge_sorted_topk(tops):
    top16 = jnp.sort(tops[0])
    for k in range(1, K):
        desc = jnp.flip(jnp.sort(tops[k]))
        top16 = jnp.sort(jnp.maximum(top16, desc))
    return top16


@functools.partial(
    pl.kernel,
    out_type=(
        jax.ShapeDtypeStruct((NW, L), jnp.float32),
        jax.ShapeDtypeStruct((NW, L), jnp.float32),
    ),
    mesh=_mesh,
    compiler_params=_params,
    scratch_types=[
        pltpu.VMEM((PW,), jnp.float32),
        pltpu.VMEM((PW,), jnp.float32),
        pltpu.VMEM((L,), jnp.float32),
        pltpu.VMEM((L,), jnp.float32),
    ],
)
def _scan_kernel(c0_hbm, c1_hbm, cand_hbm, psum_hbm, buf0, buf1, candv, psumv):
    wid = lax.axis_index("s") * NC + lax.axis_index("c")
    base = wid * PW

    pltpu.sync_copy(c0_hbm.at[pl.ds(base, PW_LAST)], buf0.at[pl.ds(0, PW_LAST)])
    pltpu.sync_copy(c1_hbm.at[pl.ds(base, PW_LAST)], buf1.at[pl.ds(0, PW_LAST)])

    @pl.when(wid < NW - 1)
    def _():
        pltpu.sync_copy(c0_hbm.at[pl.ds(base + PW_LAST, PW - PW_LAST)],
                        buf0.at[pl.ds(PW_LAST, PW - PW_LAST)])
        pltpu.sync_copy(c1_hbm.at[pl.ds(base + PW_LAST, PW - PW_LAST)],
                        buf1.at[pl.ds(PW_LAST, PW - PW_LAST)])

    def body(i, carry):
        acc = carry[0]
        tops = list(carry[1:])
        start = i * L
        lengths = _sanitize(buf1[pl.ds(start, L)] - buf0[pl.ds(start, L)])
        acc = acc + lengths * lengths
        tops = _insert_top(tops, lengths)
        return (acc, *tops)

    init = (jnp.zeros((L,), jnp.float32),
            *[jnp.full((L,), NEG_INF, jnp.float32) for _ in range(K)])
    carry = lax.fori_loop(0, ITER_A, body, init)
    carry = lax.cond(wid < NW - 1,
                     lambda c: lax.fori_loop(ITER_A, ITER_A + ITER_B, body, c),
                     lambda c: c,
                     carry)

    psumv[...] = carry[0]
    candv[...] = _merge_sorted_topk(list(carry[1:]))
    pltpu.sync_copy(candv, cand_hbm.at[wid])
    pltpu.sync_copy(psumv, psum_hbm.at[wid])


@functools.partial(
    pl.kernel,
    out_type=jax.ShapeDtypeStruct((L,), jnp.float32),
    mesh=_mesh,
    compiler_params=_params,
    scratch_types=[
        pltpu.VMEM((NW * L,), jnp.float32),
        pltpu.VMEM((NW * L,), jnp.float32),
        pltpu.VMEM((L,), jnp.float32),
    ],
)
def _merge_kernel(cand_hbm, psum_hbm, out_hbm, cbuf, pbuf, outv):
    wid = lax.axis_index("s") * NC + lax.axis_index("c")

    @pl.when(wid == 0)
    def _():
        pltpu.sync_copy(cand_hbm, cbuf)
        pltpu.sync_copy(psum_hbm, pbuf)

        def sum_body(j, acc):
            return acc + pbuf[pl.ds(j * L, L)]

        totv = lax.fori_loop(0, NW, sum_body,
                             jnp.zeros((L,), jnp.float32))
        total = jnp.sum(totv)

        def top_body(i, carry):
            return tuple(_insert_top(list(carry), cbuf[pl.ds(i * L, L)]))

        init = tuple(jnp.full((L,), NEG_INF, jnp.float32) for _ in range(K))
        tops = lax.fori_loop(0, NW, top_body, init)

        top16 = _merge_sorted_topk(list(tops))
        result = total - jnp.sum(top16 * top16)
        outv[...] = jnp.full((L,), result, jnp.float32)
        pltpu.sync_copy(outv, out_hbm)


def kernel(dgm):
    c0 = dgm[:, 0]
    c1 = dgm[:, 1]
    cand, psum = _scan_kernel(c0, c1)
    out = _merge_kernel(jnp.reshape(cand, (NW * L,)),
                        jnp.reshape(psum, (NW * L,)))
    return out[0]


# speedup vs baseline: 13.0586x; 1.0027x over previous
import functools

import jax
import jax.numpy as jnp
from jax import lax
from jax.experimental import pallas as pl
from jax.experimental.pallas import tpu as pltpu
from jax.experimental.pallas import tpu_sc as plsc

N = 500000
SPLIT = 249984
NA = SPLIT
NB = N - SPLIT
K = 16
L = 16
NC = 2
NS = 16
NW = NC * NS
NEG_INF = float("-inf")

_mesh = plsc.VectorSubcoreMesh(core_axis_name="c", subcore_axis_name="s")
_params = pltpu.CompilerParams(needs_layout_passes=False,
                               use_tc_tiling_on_sc=False)


def _sanitize(lengths):
    lengths = jnp.where(jnp.isinf(lengths), jnp.float32(0.0), lengths)
    return jnp.where(jnp.isnan(lengths), jnp.float32(0.0), lengths)


def _insert_top(tops, v):
    new_tops = []
    for t in tops:
        hi = jnp.maximum(t, v)
        v = jnp.minimum(t, v)
        new_tops.append(hi)
    return new_tops


def _merge_sorted_topk(tops):
    top16 = jnp.sort(tops[0])
    for k in range(1, K):
        desc = jnp.flip(jnp.sort(tops[k]))
        top16 = jnp.sort(jnp.maximum(top16, desc))
    return top16


def _make_scan(n):
    pw = -(-n // (NW * L)) * L
    pw_last = n - (NW - 1) * pw
    assert pw % L == 0 and pw_last % L == 0 and 0 < pw_last <= pw
    iter_a = pw_last // L
    iter_b = (pw - pw_last) // L

    @functools.partial(
        pl.kernel,
        out_type=(
            jax.ShapeDtypeStruct((NW, L), jnp.float32),
            jax.ShapeDtypeStruct((NW, L), jnp.float32),
        ),
        mesh=_mesh,
        compiler_params=_params,
        scratch_types=[
            pltpu.VMEM((pw,), jnp.float32),
            pltpu.VMEM((pw,), jnp.float32),
            pltpu.VMEM((L,), jnp.float32),
            pltpu.VMEM((L,), jnp.float32),
        ],
    )
    def scan(c0_hbm, c1_hbm, cand_hbm, psum_hbm, buf0, buf1, candv, psumv):
        wid = lax.axis_index("s") * NC + lax.axis_index("c")
        base = wid * pw

        pltpu.sync_copy(c0_hbm.at[pl.ds(base, pw_last)],
                        buf0.at[pl.ds(0, pw_last)])
        pltpu.sync_copy(c1_hbm.at[pl.ds(base, pw_last)],
                        buf1.at[pl.ds(0, pw_last)])

        @pl.when(wid < NW - 1)
        def _():
            pltpu.sync_copy(c0_hbm.at[pl.ds(base + pw_last, pw - pw_last)],
                            buf0.at[pl.ds(pw_last, pw - pw_last)])
            pltpu.sync_copy(c1_hbm.at[pl.ds(base + pw_last, pw - pw_last)],
                            buf1.at[pl.ds(pw_last, pw - pw_last)])

        def body(i, carry):
            acc = carry[0]
            tops = list(carry[1:])
            start = i * L
            lengths = _sanitize(buf1[pl.ds(start, L)] - buf0[pl.ds(start, L)])
            acc = acc + lengths * lengths
            tops = _insert_top(tops, lengths)
            return (acc, *tops)

        init = (jnp.zeros((L,), jnp.float32),
                *[jnp.full((L,), NEG_INF, jnp.float32) for _ in range(K)])
        carry = lax.fori_loop(0, iter_a, body, init)
        carry = lax.cond(wid < NW - 1,
                         lambda c: lax.fori_loop(iter_a, iter_a + iter_b,
                                                 body, c),
                         lambda c: c,
                         carry)

        psumv[...] = carry[0]
        candv[...] = _merge_sorted_topk(list(carry[1:]))
        pltpu.sync_copy(candv, cand_hbm.at[wid])
        pltpu.sync_copy(psumv, psum_hbm.at[wid])

    return scan


_scan_a = _make_scan(NA)
_scan_b = _make_scan(NB)


@functools.partial(
    pl.kernel,
    out_type=jax.ShapeDtypeStruct((L,), jnp.float32),
    mesh=_mesh,
    compiler_params=_params,
    scratch_types=[
        pltpu.VMEM((NW * L,), jnp.float32),
        pltpu.VMEM((NW * L,), jnp.float32),
        pltpu.VMEM((NW * L,), jnp.float32),
        pltpu.VMEM((NW * L,), jnp.float32),
        pltpu.VMEM((L,), jnp.float32),
    ],
)
def _merge_kernel(ca_hbm, pa_hbm, cb_hbm, pb_hbm, out_hbm,
                  cabuf, pabuf, cbbuf, pbbuf, outv):
    wid = lax.axis_index("s") * NC + lax.axis_index("c")

    @pl.when(wid == 0)
    def _():
        pltpu.sync_copy(ca_hbm, cabuf)
        pltpu.sync_copy(pa_hbm, pabuf)
        pltpu.sync_copy(cb_hbm, cbbuf)
        pltpu.sync_copy(pb_hbm, pbbuf)

        def sum_body(j, acc):
            return acc + pabuf[pl.ds(j * L, L)] + pbbuf[pl.ds(j * L, L)]

        totv = lax.fori_loop(0, NW, sum_body, jnp.zeros((L,), jnp.float32))
        total = jnp.sum(totv)

        def top_body(j, carry):
            tops = _insert_top(list(carry), cabuf[pl.ds(j * L, L)])
            return tuple(_insert_top(tops, cbbuf[pl.ds(j * L, L)]))

        init = tuple(jnp.full((L,), NEG_INF, jnp.float32) for _ in range(K))
        tops = lax.fori_loop(0, NW, top_body, init)

        top16 = _merge_sorted_topk(list(tops))
        result = total - jnp.sum(top16 * top16)
        outv[...] = jnp.full((L,), result, jnp.float32)
        pltpu.sync_copy(outv, out_hbm)


def kernel(dgm):
    c0a = dgm[:SPLIT, 0]
    c1a = dgm[:SPLIT, 1]
    cand_a, psum_a = _scan_a(c0a, c1a)
    c0b = dgm[SPLIT:, 0]
    c1b = dgm[SPLIT:, 1]
    cand_b, psum_b = _scan_b(c0b, c1b)
    out = _merge_kernel(jnp.reshape(cand_a, (NW * L,)),
                        jnp.reshape(psum_a, (NW * L,)),
                        jnp.reshape(cand_b, (NW * L,)),
                        jnp.reshape(psum_b, (NW * L,)))
    return out[0]


# speedup vs baseline: 13.6734x; 1.0471x over previous
import functools

import jax
import jax.numpy as jnp
from jax import lax
from jax.experimental import pallas as pl
from jax.experimental.pallas import tpu as pltpu
from jax.experimental.pallas import tpu_sc as plsc

N = 500000
NA = 228608
NB = 154624
ND = N - NA - NB
K = 16
L = 16
NC = 2
NS = 16
NW = NC * NS
NEG_INF = float("-inf")

_mesh = plsc.VectorSubcoreMesh(core_axis_name="c", subcore_axis_name="s")
_params = pltpu.CompilerParams(needs_layout_passes=False,
                               use_tc_tiling_on_sc=False)


def _sanitize(lengths):
    lengths = jnp.where(jnp.isinf(lengths), jnp.float32(0.0), lengths)
    return jnp.where(jnp.isnan(lengths), jnp.float32(0.0), lengths)


def _insert_top(tops, v):
    new_tops = []
    for t in tops:
        hi = jnp.maximum(t, v)
        v = jnp.minimum(t, v)
        new_tops.append(hi)
    return new_tops


def _merge_sorted_topk(tops):
    top16 = jnp.sort(tops[0])
    for k in range(1, K):
        desc = jnp.flip(jnp.sort(tops[k]))
        top16 = jnp.sort(jnp.maximum(top16, desc))
    return top16


def _make_scan(n):
    pw = -(-n // (NW * L)) * L
    pw_last = n - (NW - 1) * pw
    assert pw % L == 0 and pw_last % L == 0 and 0 < pw_last <= pw
    iter_a = pw_last // L
    iter_b = (pw - pw_last) // L

    @functools.partial(
        pl.kernel,
        out_type=(
            jax.ShapeDtypeStruct((NW, L), jnp.float32),
            jax.ShapeDtypeStruct((NW, L), jnp.float32),
        ),
        mesh=_mesh,
        compiler_params=_params,
        scratch_types=[
            pltpu.VMEM((pw,), jnp.float32),
            pltpu.VMEM((pw,), jnp.float32),
            pltpu.VMEM((L,), jnp.float32),
            pltpu.VMEM((L,), jnp.float32),
            pltpu.SemaphoreType.DMA,
        ],
    )
    def scan(c0_hbm, c1_hbm, cand_hbm, psum_hbm, buf0, buf1, candv, psumv,
             dsem):
        wid = lax.axis_index("s") * NC + lax.axis_index("c")
        base = wid * pw

        cp0 = pltpu.async_copy(c0_hbm.at[pl.ds(base, pw_last)],
                               buf0.at[pl.ds(0, pw_last)], dsem)
        cp1 = pltpu.async_copy(c1_hbm.at[pl.ds(base, pw_last)],
                               buf1.at[pl.ds(0, pw_last)], dsem)
        if pw != pw_last:
            @pl.when(wid < NW - 1)
            def _():
                cpt0 = pltpu.async_copy(
                    c0_hbm.at[pl.ds(base + pw_last, pw - pw_last)],
                    buf0.at[pl.ds(pw_last, pw - pw_last)], dsem)
                cpt1 = pltpu.async_copy(
                    c1_hbm.at[pl.ds(base + pw_last, pw - pw_last)],
                    buf1.at[pl.ds(pw_last, pw - pw_last)], dsem)
                cpt0.wait()
                cpt1.wait()
        cp0.wait()
        cp1.wait()

        def body(i, carry):
            acc = carry[0]
            tops = list(carry[1:])
            start = i * L
            lengths = _sanitize(buf1[pl.ds(start, L)] - buf0[pl.ds(start, L)])
            acc = acc + lengths * lengths
            tops = _insert_top(tops, lengths)
            return (acc, *tops)

        init = (jnp.zeros((L,), jnp.float32),
                *[jnp.full((L,), NEG_INF, jnp.float32) for _ in range(K)])
        carry = lax.fori_loop(0, iter_a, body, init)
        if iter_b:
            carry = lax.cond(wid < NW - 1,
                             lambda c: lax.fori_loop(iter_a, iter_a + iter_b,
                                                     body, c),
                             lambda c: c,
                             carry)

        psumv[...] = carry[0]
        candv[...] = _merge_sorted_topk(list(carry[1:]))
        pltpu.sync_copy(candv, cand_hbm.at[wid])
        pltpu.sync_copy(psumv, psum_hbm.at[wid])

    return scan


_scan_a = _make_scan(NA)
_scan_b = _make_scan(NB)
_scan_d = _make_scan(ND)

NCHUNK = 3


@functools.partial(
    pl.kernel,
    out_type=jax.ShapeDtypeStruct((L,), jnp.float32),
    mesh=_mesh,
    compiler_params=_params,
    scratch_types=[
        pltpu.VMEM((NCHUNK * NW * L,), jnp.float32),
        pltpu.VMEM((NCHUNK * NW * L,), jnp.float32),
        pltpu.VMEM((L,), jnp.float32),
        pltpu.SemaphoreType.DMA,
    ],
)
def _merge_kernel(ca_hbm, pa_hbm, cb_hbm, pb_hbm, cd_hbm, pd_hbm, out_hbm,
                  cbuf, pbuf, outv, dsem):
    wid = lax.axis_index("s") * NC + lax.axis_index("c")

    @pl.when(wid == 0)
    def _():
        cps = [
            pltpu.async_copy(ca_hbm, cbuf.at[pl.ds(0, NW * L)], dsem),
            pltpu.async_copy(cb_hbm, cbuf.at[pl.ds(NW * L, NW * L)], dsem),
            pltpu.async_copy(cd_hbm, cbuf.at[pl.ds(2 * NW * L, NW * L)], dsem),
            pltpu.async_copy(pa_hbm, pbuf.at[pl.ds(0, NW * L)], dsem),
            pltpu.async_copy(pb_hbm, pbuf.at[pl.ds(NW * L, NW * L)], dsem),
            pltpu.async_copy(pd_hbm, pbuf.at[pl.ds(2 * NW * L, NW * L)], dsem),
        ]
        for cp in cps:
            cp.wait()

        def sum_body(j, acc):
            return acc + pbuf[pl.ds(j * L, L)]

        totv = lax.fori_loop(0, NCHUNK * NW, sum_body,
                             jnp.zeros((L,), jnp.float32))
        total = jnp.sum(totv)

        def top_body(j, carry):
            return tuple(_insert_top(list(carry), cbuf[pl.ds(j * L, L)]))

        init = tuple(jnp.full((L,), NEG_INF, jnp.float32) for _ in range(K))
        tops = lax.fori_loop(0, NCHUNK * NW, top_body, init)

        top16 = _merge_sorted_topk(list(tops))
        result = total - jnp.sum(top16 * top16)
        outv[...] = jnp.full((L,), result, jnp.float32)
        pltpu.sync_copy(outv, out_hbm)


def kernel(dgm):
    cand_a, psum_a = _scan_a(dgm[:NA, 0], dgm[:NA, 1])
    cand_b, psum_b = _scan_b(dgm[NA:NA + NB, 0], dgm[NA:NA + NB, 1])
    cand_d, psum_d = _scan_d(dgm[NA + NB:, 0], dgm[NA + NB:, 1])
    out = _merge_kernel(jnp.reshape(cand_a, (NW * L,)),
                        jnp.reshape(psum_a, (NW * L,)),
                        jnp.reshape(cand_b, (NW * L,)),
                        jnp.reshape(psum_b, (NW * L,)),
                        jnp.reshape(cand_d, (NW * L,)),
                        jnp.reshape(psum_d, (NW * L,)))
    return out[0]


# speedup vs baseline: 13.7114x; 1.0028x over previous
import functools

import jax
import jax.numpy as jnp
from jax import lax
from jax.experimental import pallas as pl
from jax.experimental.pallas import tpu as pltpu
from jax.experimental.pallas import tpu_sc as plsc

N = 500000
NA = 320000
NB = 150016
ND = N - NA - NB
K = 16
L = 16
NC = 2
NS = 16
NW = NC * NS
NEG_INF = float("-inf")

_mesh2 = plsc.VectorSubcoreMesh(core_axis_name="c", subcore_axis_name="s")
_mesh1 = plsc.VectorSubcoreMesh(core_axis_name="c", subcore_axis_name="s",
                                num_cores=1)
_params = pltpu.CompilerParams(needs_layout_passes=False,
                               use_tc_tiling_on_sc=False)


def _sanitize(lengths):
    lengths = jnp.where(jnp.isinf(lengths), jnp.float32(0.0), lengths)
    return jnp.where(jnp.isnan(lengths), jnp.float32(0.0), lengths)


def _insert_top(tops, v):
    new_tops = []
    for t in tops:
        hi = jnp.maximum(t, v)
        v = jnp.minimum(t, v)
        new_tops.append(hi)
    return new_tops


def _merge_sorted_topk(tops):
    top16 = jnp.sort(tops[0])
    for k in range(1, K):
        desc = jnp.flip(jnp.sort(tops[k]))
        top16 = jnp.sort(jnp.maximum(top16, desc))
    return top16


def _stage_chunk(c0_hbm, c1_hbm, buf0, buf1, dsem, base, pw, pw_last, wid,
                 n_workers):
    cp0 = pltpu.async_copy(c0_hbm.at[pl.ds(base, pw_last)],
                           buf0.at[pl.ds(0, pw_last)], dsem)
    cp1 = pltpu.async_copy(c1_hbm.at[pl.ds(base, pw_last)],
                           buf1.at[pl.ds(0, pw_last)], dsem)
    if pw != pw_last:
        @pl.when(wid < n_workers - 1)
        def _():
            cpt0 = pltpu.async_copy(
                c0_hbm.at[pl.ds(base + pw_last, pw - pw_last)],
                buf0.at[pl.ds(pw_last, pw - pw_last)], dsem)
            cpt1 = pltpu.async_copy(
                c1_hbm.at[pl.ds(base + pw_last, pw - pw_last)],
                buf1.at[pl.ds(pw_last, pw - pw_last)], dsem)
            cpt0.wait()
            cpt1.wait()
    cp0.wait()
    cp1.wait()


def _scan_chunk(buf0, buf1, iter_a, iter_b, wid, n_workers):
    def body(i, carry):
        acc = carry[0]
        tops = list(carry[1:])
        start = i * L
        lengths = _sanitize(buf1[pl.ds(start, L)] - buf0[pl.ds(start, L)])
        acc = acc + lengths * lengths
        tops = _insert_top(tops, lengths)
        return (acc, *tops)

    init = (jnp.zeros((L,), jnp.float32),
            *[jnp.full((L,), NEG_INF, jnp.float32) for _ in range(K)])
    carry = lax.fori_loop(0, iter_a, body, init)
    if iter_b:
        carry = lax.cond(wid < n_workers - 1,
                         lambda c: lax.fori_loop(iter_a, iter_a + iter_b,
                                                 body, c),
                         lambda c: c,
                         carry)
    return carry[0], _merge_sorted_topk(list(carry[1:]))


def _split(n, n_workers):
    pw = -(-n // (n_workers * L)) * L
    pw_last = n - (n_workers - 1) * pw
    assert pw % L == 0 and pw_last % L == 0 and 0 < pw_last <= pw
    return pw, pw_last


def _make_scan(n):
    pw, pw_last = _split(n, NW)

    @functools.partial(
        pl.kernel,
        out_type=(
            jax.ShapeDtypeStruct((NW, L), jnp.float32),
            jax.ShapeDtypeStruct((NW, L), jnp.float32),
        ),
        mesh=_mesh2,
        compiler_params=_params,
        scratch_types=[
            pltpu.VMEM((pw,), jnp.float32),
            pltpu.VMEM((pw,), jnp.float32),
            pltpu.VMEM((L,), jnp.float32),
            pltpu.VMEM((L,), jnp.float32),
            pltpu.SemaphoreType.DMA,
        ],
    )
    def scan(c0_hbm, c1_hbm, cand_hbm, psum_hbm, buf0, buf1, candv, psumv,
             dsem):
        wid = lax.axis_index("s") * NC + lax.axis_index("c")
        _stage_chunk(c0_hbm, c1_hbm, buf0, buf1, dsem, wid * pw, pw, pw_last,
                     wid, NW)
        acc, top16 = _scan_chunk(buf0, buf1, pw_last // L,
                                 (pw - pw_last) // L, wid, NW)
        psumv[...] = acc
        candv[...] = top16
        pltpu.sync_copy(candv, cand_hbm.at[wid])
        pltpu.sync_copy(psumv, psum_hbm.at[wid])

    return scan


_scan_a = _make_scan(NA)
_scan_b = _make_scan(NB)

_PWD, _PWD_LAST = _split(ND, NS)


@functools.partial(
    pl.kernel,
    out_type=jax.ShapeDtypeStruct((L,), jnp.float32),
    mesh=_mesh1,
    compiler_params=_params,
    scratch_types=[
        pltpu.VMEM((_PWD,), jnp.float32),
        pltpu.VMEM((_PWD,), jnp.float32),
        pltpu.VMEM((2 * NW * L,), jnp.float32),
        pltpu.VMEM((2 * NW * L,), jnp.float32),
        pltpu.VMEM((NS, 2 * L), jnp.float32),
        pltpu.VMEM_SHARED((NS, 2 * L), jnp.float32),
        pltpu.VMEM((2 * L,), jnp.float32),
        pltpu.VMEM((L,), jnp.float32),
        pltpu.SemaphoreType.DMA,
        pltpu.SemaphoreType.DMA,
    ],
)
def _final_kernel(c0_hbm, c1_hbm, ca_hbm, pa_hbm, cb_hbm, pb_hbm, out_hbm,
                  buf0, buf1, abbuf, psbuf, locbuf, shared, pubv, outv,
                  dsem, psem):
    wid = lax.axis_index("s")

    @pl.when(wid == 0)
    def _():
        pltpu.async_copy(ca_hbm, abbuf.at[pl.ds(0, NW * L)], psem)
        pltpu.async_copy(cb_hbm, abbuf.at[pl.ds(NW * L, NW * L)], psem)
        pltpu.async_copy(pa_hbm, psbuf.at[pl.ds(0, NW * L)], psem)
        pltpu.async_copy(pb_hbm, psbuf.at[pl.ds(NW * L, NW * L)], psem)

    _stage_chunk(c0_hbm, c1_hbm, buf0, buf1, dsem, wid * _PWD, _PWD,
                 _PWD_LAST, wid, NS)
    acc, top16 = _scan_chunk(buf0, buf1, _PWD_LAST // L,
                             (_PWD - _PWD_LAST) // L, wid, NS)

    pubv[pl.ds(0, L)] = top16
    pubv[pl.ds(L, L)] = acc
    pltpu.sync_copy(pubv, shared.at[wid])
    plsc.subcore_barrier()

    @pl.when(wid == 0)
    def _():
        pltpu.sync_copy(shared, locbuf)
        for j in range(4):
            pltpu.make_async_copy(ca_hbm, abbuf.at[pl.ds(0, NW * L)],
                                  psem).wait()

        def sum_body(j, acc2):
            return acc2 + psbuf[pl.ds(j * L, L)]

        totv = lax.fori_loop(0, 2 * NW, sum_body,
                             jnp.zeros((L,), jnp.float32))

        def lsum_body(j, acc2):
            return acc2 + locbuf[j, pl.ds(L, L)]

        totv = lax.fori_loop(0, NS, lsum_body, totv)
        total = jnp.sum(totv)

        def top_body(j, carry):
            return tuple(_insert_top(list(carry), abbuf[pl.ds(j * L, L)]))

        init = tuple(jnp.full((L,), NEG_INF, jnp.float32) for _ in range(K))
        tops = lax.fori_loop(0, 2 * NW, top_body, init)

        def ltop_body(j, carry):
            return tuple(_insert_top(list(carry), locbuf[j, pl.ds(0, L)]))

        tops = lax.fori_loop(0, NS, ltop_body, tops)

        top16g = _merge_sorted_topk(list(tops))
        result = total - jnp.sum(top16g * top16g)
        outv[...] = jnp.full((L,), result, jnp.float32)
        pltpu.sync_copy(outv, out_hbm)


def kernel(dgm):
    cand_a, psum_a = _scan_a(dgm[:NA, 0], dgm[:NA, 1])
    cand_b, psum_b = _scan_b(dgm[NA:NA + NB, 0], dgm[NA:NA + NB, 1])
    out = _final_kernel(dgm[NA + NB:, 0], dgm[NA + NB:, 1],
                        jnp.reshape(cand_a, (NW * L,)),
                        jnp.reshape(psum_a, (NW * L,)),
                        jnp.reshape(cand_b, (NW * L,)),
                        jnp.reshape(psum_b, (NW * L,)))
    return out[0]
